# baseline (speedup 1.0000x reference)
import functools

import jax
import jax.numpy as jnp
from jax import lax
from jax.experimental import pallas as pl
from jax.experimental.pallas import tpu as pltpu
from jax.experimental.pallas import tpu_sc as plsc

_D = 256
_BA = 4
_SEQ = 2048
_B = _BA * _SEQ
_NC = 2
_NS = 16
_NW = _NC * _NS
_BPW = _B // _NW
_WPR = _SEQ // _BPW

_mesh = plsc.VectorSubcoreMesh(core_axis_name="c", subcore_axis_name="s")


@functools.partial(
    pl.kernel,
    mesh=_mesh,
    compiler_params=pltpu.CompilerParams(needs_layout_passes=False),
    out_type=jax.ShapeDtypeStruct((_BA, 16, 16, _SEQ), jnp.float32),
    scratch_types=[
        pltpu.VMEM((_BPW,), jnp.int32),
        pltpu.VMEM((96, _D), jnp.float32),
        pltpu.VMEM((128, _D), jnp.float32),
        pltpu.VMEM((16, 16, 128), jnp.float32),
        pltpu.VMEM((16, 16, 128), jnp.float32),
        pltpu.SemaphoreType.DMA,
        pltpu.SemaphoreType.DMA,
        pltpu.SemaphoreType.DMA,
        pltpu.SemaphoreType.DMA,
        pltpu.SemaphoreType.DMA,
    ],
)
def _gather_rows(
    idx_hbm, table_hbm, out_hbm, idx_v, rows0, rows1, tr0, tr1, g0, g1, g2, w0, w1
):
    wid = lax.axis_index("s") * _NC + lax.axis_index("c")
    b = wid // _WPR
    t0 = (wid % _WPR) * _BPW
    pltpu.sync_copy(idx_hbm.at[b, pl.ds(t0, _BPW)], idx_v)

    lane = lax.iota(jnp.int32, 16)
    perms = {s: (lane ^ s).reshape(16, 1) for s in (1, 2, 4, 8)}
    masks = {s: (lane & s) == 0 for s in (1, 2, 4, 8)}
    _dnums = lax.GatherDimensionNumbers(
        offset_dims=(), collapsed_slice_dims=(0,), start_index_map=(0,)
    )

    def _xlane(x, s):
        return lax.gather(
            x, perms[s], _dnums, (1,), mode=lax.GatherScatterMode.PROMISE_IN_BOUNDS
        )

    def transpose16(vs):
        for s in (1, 2, 4, 8):
            out = list(vs)
            for k in range(16):
                if k & s == 0:
                    a, bv = vs[k], vs[k | s]
                    out[k] = jnp.where(masks[s], a, _xlane(bv, s))
                    out[k | s] = jnp.where(masks[s], _xlane(a, s), bv)
            vs = out
        return vs

    def transpose_run(rows_v, src_off, trans_v, dst_off, nblocks):
        def block(sb):
            s0 = sb * 16
            for i in range(16):
                vs = [
                    rows_v[src_off + s0 + k, pl.ds(16 * i, 16)] for k in range(16)
                ]
                vt = transpose16(vs)
                for j in range(16):
                    trans_v[i, j, pl.ds(dst_off + s0, 16)] = vt[j]

        plsc.parallel_loop(0, nblocks)(block)

    c0 = pltpu.async_copy(table_hbm.at[idx_v.at[pl.ds(0, 96)]], rows0, g0)
    c1 = pltpu.async_copy(table_hbm.at[idx_v.at[pl.ds(96, 128)]], rows1, g1)

    c0.wait()
    transpose_run(rows0, 0, tr0, 0, 6)
    c2 = pltpu.async_copy(
        table_hbm.at[idx_v.at[pl.ds(224, 32)]], rows0.at[pl.ds(0, 32)], g2
    )
    c1.wait()
    transpose_run(rows1, 0, tr0, 96, 2)
    wc0 = pltpu.async_copy(
        tr0, out_hbm.at[b, pl.ds(0, 16), pl.ds(0, 16), pl.ds(t0, 128)], w0
    )
    transpose_run(rows1, 32, tr1, 0, 6)
    c2.wait()
    transpose_run(rows0, 0, tr1, 96, 2)
    wc1 = pltpu.async_copy(
        tr1, out_hbm.at[b, pl.ds(0, 16), pl.ds(0, 16), pl.ds(t0 + 128, 128)], w1
    )
    wc0.wait()
    wc1.wait()


def kernel(input_ids, embedding):
    out = _gather_rows(input_ids.astype(jnp.int32), embedding)
    return (jnp.transpose(out, (0, 3, 1, 2)),)


# speedup vs baseline: 1.0819x; 1.0819x over previous
import functools

import jax
import jax.numpy as jnp
from jax import lax
from jax.experimental import pallas as pl
from jax.experimental.pallas import tpu as pltpu
from jax.experimental.pallas import tpu_sc as plsc

_D = 256
_BA = 4
_SEQ = 2048
_B = _BA * _SEQ
_NC = 2
_NS = 16
_NW = _NC * _NS
_BPW = _B // _NW
_CH = 128
_NCH = _BPW // _CH
_WPR = _SEQ // _BPW

_mesh = plsc.VectorSubcoreMesh(core_axis_name="c", subcore_axis_name="s")


@functools.partial(
    pl.kernel,
    mesh=_mesh,
    compiler_params=pltpu.CompilerParams(needs_layout_passes=False),
    out_type=jax.ShapeDtypeStruct((_BA, 16, 16, _SEQ), jnp.float32),
    scratch_types=[
        pltpu.VMEM((_BPW,), jnp.int32),
        pltpu.VMEM((_CH, _D), jnp.float32),
        pltpu.VMEM((_CH, _D), jnp.float32),
        pltpu.VMEM((16, 16, _CH), jnp.float32),
        pltpu.SemaphoreType.DMA,
        pltpu.SemaphoreType.DMA,
    ],
)
def _gather_rows(idx_hbm, table_hbm, out_hbm, idx_v, rows0, rows1, trans_v, s0, s1):
    wid = lax.axis_index("s") * _NC + lax.axis_index("c")
    b = wid // _WPR
    t0 = (wid % _WPR) * _BPW
    pltpu.sync_copy(idx_hbm.at[b, pl.ds(t0, _BPW)], idx_v)
    rows = [rows0, rows1]
    sems = [s0, s1]
    copies = [
        pltpu.async_copy(
            table_hbm.at[idx_v.at[pl.ds(c * _CH, _CH)]], rows[c], sems[c]
        )
        for c in range(_NCH)
    ]

    lane = lax.iota(jnp.int32, 16)
    perms = {s: (lane ^ s).reshape(16, 1) for s in (1, 2, 4, 8)}
    masks = {s: (lane & s) == 0 for s in (1, 2, 4, 8)}
    _dnums = lax.GatherDimensionNumbers(
        offset_dims=(), collapsed_slice_dims=(0,), start_index_map=(0,)
    )

    def _xlane(x, s):
        return lax.gather(
            x,
            perms[s],
            _dnums,
            (1,),
            mode=lax.GatherScatterMode.PROMISE_IN_BOUNDS,
        )

    def transpose16(vs):
        for s in (1, 2, 4, 8):
            out = list(vs)
            for k in range(16):
                if k & s == 0:
                    a, bv = vs[k], vs[k | s]
                    out[k] = jnp.where(masks[s], a, _xlane(bv, s))
                    out[k | s] = jnp.where(masks[s], _xlane(a, s), bv)
            vs = out
        return vs

    for c in range(_NCH):
        rows_v = rows[c]

        def transpose_block(sb):
            s0 = sb * 16
            for i in range(16):
                vs = [rows_v[s0 + k, pl.ds(16 * i, 16)] for k in range(16)]
                vt = transpose16(vs)
                for j in range(16):
                    trans_v[i, j, pl.ds(s0, 16)] = vt[j]

        copies[c].wait()
        plsc.parallel_loop(0, _CH // 16)(transpose_block)
        pltpu.sync_copy(
            trans_v,
            out_hbm.at[b, pl.ds(0, 16), pl.ds(0, 16), pl.ds(t0 + c * _CH, _CH)],
        )


def kernel(input_ids, embedding):
    out = _gather_rows(input_ids.astype(jnp.int32), embedding)
    return (jnp.transpose(out, (0, 3, 1, 2)),)
